# baseline (speedup 1.0000x reference)
import functools

import jax
import jax.numpy as jnp
from jax import lax
from jax.experimental import pallas as pl
from jax.experimental.pallas import tpu as pltpu
from jax.experimental.pallas import tpu_sc as plsc

D_MODEL = 64
WIDE = 128
VOCAB_CHUNK = 2048
LANES = 128
K = 4
NC = 2
NS = 16
NW = NC * NS


def _transpose_block(in_ref, out_ref):
    vt = in_ref[...].T
    v3 = vt.reshape(VOCAB_CHUNK // 2, 2, D_MODEL)
    out_ref[...] = jnp.concatenate([v3[:, 0, :], v3[:, 1, :]], axis=1)


@functools.lru_cache(maxsize=None)
def _build_transpose(V):
    grid = (V + VOCAB_CHUNK - 1) // VOCAB_CHUNK
    return pl.pallas_call(
        _transpose_block,
        grid=(grid,),
        in_specs=[pl.BlockSpec((D_MODEL, VOCAB_CHUNK), lambda i: (0, i))],
        out_specs=pl.BlockSpec((VOCAB_CHUNK // 2, WIDE), lambda i: (i, 0)),
        out_shape=jax.ShapeDtypeStruct((V // 2, WIDE), jnp.float32),
    )


@functools.lru_cache(maxsize=None)
def _build_gather(B, V2):
    rows_total = B // LANES
    rows_per_w = rows_total // NW
    n_steps = rows_per_w // K
    T = n_steps // 2
    CG = K * LANES

    mesh = plsc.VectorSubcoreMesh(core_axis_name="c", subcore_axis_name="s")

    @functools.partial(
        pl.kernel,
        mesh=mesh,
        out_type=jax.ShapeDtypeStruct((B, WIDE), jnp.float32),
        scratch_types=[
            pltpu.VMEM((rows_per_w, LANES), jnp.int32),
            pltpu.VMEM((CG, D_MODEL), jnp.float32),
            pltpu.VMEM((CG, D_MODEL), jnp.float32),
            pltpu.SemaphoreType.DMA,
            pltpu.SemaphoreType.DMA,
            pltpu.SemaphoreType.DMA,
            pltpu.SemaphoreType.DMA,
        ],
        compiler_params=pltpu.CompilerParams(use_tc_tiling_on_sc=False),
    )
    def gather_kernel(idx_hbm, table_hbm, out_hbm, idx_all, rows0, rows1,
                      sem_g0, sem_g1, sem_s0, sem_s1):
        wid = lax.axis_index("s") * NC + lax.axis_index("c")
        row_base = wid * rows_per_w
        out_base = row_base * LANES

        pltpu.sync_copy(idx_hbm.at[pl.ds(row_base, rows_per_w), :], idx_all)

        def fire_gathers(step, rows_v, sem):
            for j in range(K):
                pltpu.async_copy(
                    table_hbm.at[idx_all.at[step * K + j]],
                    rows_v.at[pl.ds(j * LANES, LANES), :],
                    sem,
                )

        def wait_gathers(rows_v, sem):
            for j in range(K):
                pltpu.make_async_copy(
                    table_hbm.at[idx_all.at[j]],
                    rows_v.at[pl.ds(j * LANES, LANES), :],
                    sem,
                ).wait()

        def fire_store(step, rows_v, sem):
            pltpu.async_copy(
                rows_v,
                out_hbm.at[pl.ds(out_base + step * CG, CG), pl.ds(0, D_MODEL)],
                sem,
            )

        def wait_store(step, rows_v, sem):
            pltpu.make_async_copy(
                rows_v,
                out_hbm.at[pl.ds(out_base + step * CG, CG), pl.ds(0, D_MODEL)],
                sem,
            ).wait()

        fire_gathers(0, rows0, sem_g0)

        def body(t, carry):
            i0 = 2 * t
            wait_gathers(rows0, sem_g0)
            fire_store(i0, rows0, sem_s0)

            @pl.when(t > 0)
            def _():
                wait_store(i0 - 1, rows1, sem_s1)

            fire_gathers(i0 + 1, rows1, sem_g1)

            wait_gathers(rows1, sem_g1)
            fire_store(i0 + 1, rows1, sem_s1)
            wait_store(i0, rows0, sem_s0)

            @pl.when(t < T - 1)
            def _():
                fire_gathers(i0 + 2, rows0, sem_g0)

            return carry

        lax.fori_loop(0, T, body, 0)
        wait_store(n_steps - 1, rows1, sem_s1)

    return gather_kernel


def kernel(x, table):
    B0, B1 = x.shape
    B = B0 * B1
    V = table.shape[0]
    idx = x.astype(jnp.int32).reshape(B // LANES, LANES)
    staged = _build_transpose(V)(table.T)
    table_lin = staged.reshape(V, D_MODEL)
    out = _build_gather(B, V)(idx, table_lin)
    return out[:, :D_MODEL].reshape(B0, B1, D_MODEL)


# speedup vs baseline: 1.4320x; 1.4320x over previous
import functools

import jax
import jax.numpy as jnp
from jax import lax
from jax.experimental import pallas as pl
from jax.experimental.pallas import tpu as pltpu
from jax.experimental.pallas import tpu_sc as plsc

D_MODEL = 64
WIDE = 128
VOCAB_CHUNK = 8192
LANES = 128
K = 4
NC = 2
NS = 16
NW = NC * NS


def _transpose_block(in_ref, out_ref):
    out_ref[:, 0:D_MODEL] = in_ref[...].T


@functools.lru_cache(maxsize=None)
def _build_transpose(V):
    grid = (V + VOCAB_CHUNK - 1) // VOCAB_CHUNK
    return pl.pallas_call(
        _transpose_block,
        grid=(grid,),
        in_specs=[pl.BlockSpec((D_MODEL, VOCAB_CHUNK), lambda i: (0, i))],
        out_specs=pl.BlockSpec((VOCAB_CHUNK, WIDE), lambda i: (i, 0)),
        out_shape=jax.ShapeDtypeStruct((V, WIDE), jnp.float32),
    )


@functools.lru_cache(maxsize=None)
def _build_gather(B, V2):
    rows_total = B // LANES
    rows_per_w = rows_total // NW
    n_steps = rows_per_w // K
    T = n_steps // 2
    CG = K * LANES

    mesh = plsc.VectorSubcoreMesh(core_axis_name="c", subcore_axis_name="s")

    @functools.partial(
        pl.kernel,
        mesh=mesh,
        out_type=jax.ShapeDtypeStruct((B, WIDE), jnp.float32),
        scratch_types=[
            pltpu.VMEM((rows_per_w, LANES), jnp.int32),
            pltpu.VMEM((CG, D_MODEL), jnp.float32),
            pltpu.VMEM((CG, D_MODEL), jnp.float32),
            pltpu.SemaphoreType.DMA,
            pltpu.SemaphoreType.DMA,
            pltpu.SemaphoreType.DMA,
            pltpu.SemaphoreType.DMA,
        ],
        compiler_params=pltpu.CompilerParams(use_tc_tiling_on_sc=False),
    )
    def gather_kernel(idx_hbm, table_hbm, out_hbm, idx_all, rows0, rows1,
                      sem_g0, sem_g1, sem_s0, sem_s1):
        wid = lax.axis_index("s") * NC + lax.axis_index("c")
        row_base = wid * rows_per_w
        out_base = row_base * LANES

        pltpu.sync_copy(idx_hbm.at[pl.ds(row_base, rows_per_w), :], idx_all)

        def fire_gathers(step, rows_v, sem):
            for j in range(K):
                pltpu.async_copy(
                    table_hbm.at[idx_all.at[step * K + j]],
                    rows_v.at[pl.ds(j * LANES, LANES), :],
                    sem,
                )

        def wait_gathers(rows_v, sem):
            for j in range(K):
                pltpu.make_async_copy(
                    table_hbm.at[idx_all.at[j]],
                    rows_v.at[pl.ds(j * LANES, LANES), :],
                    sem,
                ).wait()

        def fire_store(step, rows_v, sem):
            pltpu.async_copy(
                rows_v,
                out_hbm.at[pl.ds(out_base + step * CG, CG), pl.ds(0, D_MODEL)],
                sem,
            )

        def wait_store(step, rows_v, sem):
            pltpu.make_async_copy(
                rows_v,
                out_hbm.at[pl.ds(out_base + step * CG, CG), pl.ds(0, D_MODEL)],
                sem,
            ).wait()

        fire_gathers(0, rows0, sem_g0)

        def body(t, carry):
            i0 = 2 * t
            wait_gathers(rows0, sem_g0)
            fire_store(i0, rows0, sem_s0)

            @pl.when(t > 0)
            def _():
                wait_store(i0 - 1, rows1, sem_s1)

            fire_gathers(i0 + 1, rows1, sem_g1)

            wait_gathers(rows1, sem_g1)
            fire_store(i0 + 1, rows1, sem_s1)
            wait_store(i0, rows0, sem_s0)

            @pl.when(t < T - 1)
            def _():
                fire_gathers(i0 + 2, rows0, sem_g0)

            return carry

        lax.fori_loop(0, T, body, 0)
        wait_store(n_steps - 1, rows1, sem_s1)

    return gather_kernel


def kernel(x, table):
    B0, B1 = x.shape
    B = B0 * B1
    V = table.shape[0]
    idx2 = (x.astype(jnp.int32) * 2).reshape(B // LANES, LANES)
    staged = _build_transpose(V)(table.T)
    table_lin = staged.reshape(2 * V, D_MODEL)
    out = _build_gather(B, 2 * V)(idx2, table_lin)
    return out[:, :D_MODEL].reshape(B0, B1, D_MODEL)


# speedup vs baseline: 1.4818x; 1.0348x over previous
import functools

import jax
import jax.numpy as jnp
from jax import lax
from jax.experimental import pallas as pl
from jax.experimental.pallas import tpu as pltpu
from jax.experimental.pallas import tpu_sc as plsc

D_MODEL = 64
WIDE = 128
VOCAB_CHUNK = 16384
LANES = 128
K = 4
NC = 2
NS = 16
NW = NC * NS


def _transpose_block(in_ref, out_ref):
    out_ref[:, 0:D_MODEL] = in_ref[...].T


@functools.lru_cache(maxsize=None)
def _build_transpose(V):
    grid = (V + VOCAB_CHUNK - 1) // VOCAB_CHUNK
    return pl.pallas_call(
        _transpose_block,
        grid=(grid,),
        in_specs=[pl.BlockSpec((D_MODEL, VOCAB_CHUNK), lambda i: (0, i))],
        out_specs=pl.BlockSpec((VOCAB_CHUNK, WIDE), lambda i: (i, 0)),
        out_shape=jax.ShapeDtypeStruct((V, WIDE), jnp.float32),
    )


@functools.lru_cache(maxsize=None)
def _build_gather(B, V2):
    rows_total = B // LANES
    rows_per_w = rows_total // NW
    n_steps = rows_per_w // K
    T = n_steps // 2
    CG = K * LANES

    mesh = plsc.VectorSubcoreMesh(core_axis_name="c", subcore_axis_name="s")

    @functools.partial(
        pl.kernel,
        mesh=mesh,
        out_type=jax.ShapeDtypeStruct((B, WIDE), jnp.float32),
        scratch_types=[
            pltpu.VMEM((rows_per_w, LANES), jnp.int32),
            pltpu.VMEM((CG, D_MODEL), jnp.float32),
            pltpu.VMEM((CG, D_MODEL), jnp.float32),
            pltpu.SemaphoreType.DMA,
            pltpu.SemaphoreType.DMA,
            pltpu.SemaphoreType.DMA,
            pltpu.SemaphoreType.DMA,
        ],
        compiler_params=pltpu.CompilerParams(use_tc_tiling_on_sc=False),
    )
    def gather_kernel(idx_hbm, table_hbm, out_hbm, idx_all, rows0, rows1,
                      sem_g0, sem_g1, sem_s0, sem_s1):
        wid = lax.axis_index("s") * NC + lax.axis_index("c")
        row_base = wid * rows_per_w
        out_base = row_base * LANES

        pltpu.sync_copy(idx_hbm.at[pl.ds(row_base, rows_per_w), :], idx_all)

        def fire_gathers(step, rows_v, sem):
            for j in range(K):
                pltpu.async_copy(
                    table_hbm.at[idx_all.at[step * K + j]],
                    rows_v.at[pl.ds(j * LANES, LANES), :],
                    sem,
                )

        def wait_gathers(rows_v, sem):
            for j in range(K):
                pltpu.make_async_copy(
                    table_hbm.at[idx_all.at[j]],
                    rows_v.at[pl.ds(j * LANES, LANES), :],
                    sem,
                ).wait()

        def fire_store(step, rows_v, sem):
            pltpu.async_copy(
                rows_v,
                out_hbm.at[pl.ds(out_base + step * CG, CG), pl.ds(0, D_MODEL)],
                sem,
            )

        def wait_store(step, rows_v, sem):
            pltpu.make_async_copy(
                rows_v,
                out_hbm.at[pl.ds(out_base + step * CG, CG), pl.ds(0, D_MODEL)],
                sem,
            ).wait()

        fire_gathers(0, rows0, sem_g0)

        def body(t, carry):
            i0 = 2 * t
            wait_gathers(rows0, sem_g0)
            fire_store(i0, rows0, sem_s0)

            @pl.when(t > 0)
            def _():
                wait_store(i0 - 1, rows1, sem_s1)

            fire_gathers(i0 + 1, rows1, sem_g1)

            wait_gathers(rows1, sem_g1)
            fire_store(i0 + 1, rows1, sem_s1)
            wait_store(i0, rows0, sem_s0)

            @pl.when(t < T - 1)
            def _():
                fire_gathers(i0 + 2, rows0, sem_g0)

            return carry

        lax.fori_loop(0, T, body, 0)
        wait_store(n_steps - 1, rows1, sem_s1)

    return gather_kernel


def kernel(x, table):
    B0, B1 = x.shape
    B = B0 * B1
    V = table.shape[0]
    idx2 = (x.astype(jnp.int32) * 2).reshape(B // LANES, LANES)
    staged = _build_transpose(V)(table.T)
    table_lin = staged.reshape(2 * V, D_MODEL)
    out = _build_gather(B, 2 * V)(idx2, table_lin)
    return out[:, :D_MODEL].reshape(B0, B1, D_MODEL)


# speedup vs baseline: 1.4950x; 1.0088x over previous
import functools

import jax
import jax.numpy as jnp
from jax import lax
from jax.experimental import pallas as pl
from jax.experimental.pallas import tpu as pltpu
from jax.experimental.pallas import tpu_sc as plsc

D_MODEL = 64
WIDE = 128
VOCAB_CHUNK = 32768
LANES = 128
K = 4
NC = 2
NS = 16
NW = NC * NS


def _transpose_block(in_ref, out_ref):
    out_ref[:, 0:D_MODEL] = in_ref[...].T


@functools.lru_cache(maxsize=None)
def _build_transpose(V):
    grid = (V + VOCAB_CHUNK - 1) // VOCAB_CHUNK
    return pl.pallas_call(
        _transpose_block,
        grid=(grid,),
        in_specs=[pl.BlockSpec((D_MODEL, VOCAB_CHUNK), lambda i: (0, i))],
        out_specs=pl.BlockSpec((VOCAB_CHUNK, WIDE), lambda i: (i, 0)),
        out_shape=jax.ShapeDtypeStruct((V, WIDE), jnp.float32),
    )


@functools.lru_cache(maxsize=None)
def _build_gather(B, V2):
    rows_total = B // LANES
    rows_per_w = rows_total // NW
    n_steps = rows_per_w // K
    T = n_steps // 2
    CG = K * LANES

    mesh = plsc.VectorSubcoreMesh(core_axis_name="c", subcore_axis_name="s")

    @functools.partial(
        pl.kernel,
        mesh=mesh,
        out_type=jax.ShapeDtypeStruct((B, WIDE), jnp.float32),
        scratch_types=[
            pltpu.VMEM((rows_per_w, LANES), jnp.int32),
            pltpu.VMEM((CG, D_MODEL), jnp.float32),
            pltpu.VMEM((CG, D_MODEL), jnp.float32),
            pltpu.SemaphoreType.DMA,
            pltpu.SemaphoreType.DMA,
            pltpu.SemaphoreType.DMA,
            pltpu.SemaphoreType.DMA,
        ],
        compiler_params=pltpu.CompilerParams(use_tc_tiling_on_sc=False),
    )
    def gather_kernel(idx_hbm, table_hbm, out_hbm, idx_all, rows0, rows1,
                      sem_g0, sem_g1, sem_s0, sem_s1):
        wid = lax.axis_index("s") * NC + lax.axis_index("c")
        row_base = wid * rows_per_w
        out_base = row_base * LANES

        pltpu.sync_copy(idx_hbm.at[pl.ds(row_base, rows_per_w), :], idx_all)

        def fire_gathers(step, rows_v, sem):
            for j in range(K):
                pltpu.async_copy(
                    table_hbm.at[idx_all.at[step * K + j]],
                    rows_v.at[pl.ds(j * LANES, LANES), :],
                    sem,
                )

        def wait_gathers(rows_v, sem):
            for j in range(K):
                pltpu.make_async_copy(
                    table_hbm.at[idx_all.at[j]],
                    rows_v.at[pl.ds(j * LANES, LANES), :],
                    sem,
                ).wait()

        def fire_store(step, rows_v, sem):
            pltpu.async_copy(
                rows_v,
                out_hbm.at[pl.ds(out_base + step * CG, CG), pl.ds(0, D_MODEL)],
                sem,
            )

        def wait_store(step, rows_v, sem):
            pltpu.make_async_copy(
                rows_v,
                out_hbm.at[pl.ds(out_base + step * CG, CG), pl.ds(0, D_MODEL)],
                sem,
            ).wait()

        fire_gathers(0, rows0, sem_g0)

        def body(t, carry):
            i0 = 2 * t
            wait_gathers(rows0, sem_g0)
            fire_store(i0, rows0, sem_s0)

            @pl.when(t > 0)
            def _():
                wait_store(i0 - 1, rows1, sem_s1)

            fire_gathers(i0 + 1, rows1, sem_g1)

            wait_gathers(rows1, sem_g1)
            fire_store(i0 + 1, rows1, sem_s1)
            wait_store(i0, rows0, sem_s0)

            @pl.when(t < T - 1)
            def _():
                fire_gathers(i0 + 2, rows0, sem_g0)

            return carry

        lax.fori_loop(0, T, body, 0)
        wait_store(n_steps - 1, rows1, sem_s1)

    return gather_kernel


def kernel(x, table):
    B0, B1 = x.shape
    B = B0 * B1
    V = table.shape[0]
    idx2 = (x.astype(jnp.int32) * 2).reshape(B // LANES, LANES)
    staged = _build_transpose(V)(table.T)
    table_lin = staged.reshape(2 * V, D_MODEL)
    out = _build_gather(B, 2 * V)(idx2, table_lin)
    return out[:, :D_MODEL].reshape(B0, B1, D_MODEL)


# speedup vs baseline: 1.4954x; 1.0003x over previous
import functools

import jax
import jax.numpy as jnp
from jax import lax
from jax.experimental import pallas as pl
from jax.experimental.pallas import tpu as pltpu
from jax.experimental.pallas import tpu_sc as plsc

D_MODEL = 64
WIDE = 128
VOCAB_CHUNK = 32768
LANES = 128
K = 5
NC = 2
NS = 16
NW = NC * NS


def _transpose_block(in_ref, out_ref):
    out_ref[:, 0:D_MODEL] = in_ref[...].T


@functools.lru_cache(maxsize=None)
def _build_transpose(V):
    grid = (V + VOCAB_CHUNK - 1) // VOCAB_CHUNK
    return pl.pallas_call(
        _transpose_block,
        grid=(grid,),
        in_specs=[pl.BlockSpec((D_MODEL, VOCAB_CHUNK), lambda i: (0, i))],
        out_specs=pl.BlockSpec((VOCAB_CHUNK, WIDE), lambda i: (i, 0)),
        out_shape=jax.ShapeDtypeStruct((V, WIDE), jnp.float32),
    )


@functools.lru_cache(maxsize=None)
def _build_gather(B, V2):
    rows_total = B // LANES
    rows_per_w = rows_total // NW
    n_steps = rows_per_w // K
    T = n_steps // 2
    CG = K * LANES

    mesh = plsc.VectorSubcoreMesh(core_axis_name="c", subcore_axis_name="s")

    @functools.partial(
        pl.kernel,
        mesh=mesh,
        out_type=jax.ShapeDtypeStruct((B, WIDE), jnp.float32),
        scratch_types=[
            pltpu.VMEM((rows_per_w, LANES), jnp.int32),
            pltpu.VMEM((CG, D_MODEL), jnp.float32),
            pltpu.VMEM((CG, D_MODEL), jnp.float32),
            pltpu.SemaphoreType.DMA,
            pltpu.SemaphoreType.DMA,
            pltpu.SemaphoreType.DMA,
            pltpu.SemaphoreType.DMA,
        ],
        compiler_params=pltpu.CompilerParams(use_tc_tiling_on_sc=False),
    )
    def gather_kernel(idx_hbm, table_hbm, out_hbm, idx_all, rows0, rows1,
                      sem_g0, sem_g1, sem_s0, sem_s1):
        wid = lax.axis_index("s") * NC + lax.axis_index("c")
        row_base = wid * rows_per_w
        out_base = row_base * LANES

        pltpu.sync_copy(idx_hbm.at[pl.ds(row_base, rows_per_w), :], idx_all)

        def fire_gathers(step, rows_v, sem):
            for j in range(K):
                pltpu.async_copy(
                    table_hbm.at[idx_all.at[step * K + j]],
                    rows_v.at[pl.ds(j * LANES, LANES), :],
                    sem,
                )

        def wait_gathers(rows_v, sem):
            for j in range(K):
                pltpu.make_async_copy(
                    table_hbm.at[idx_all.at[j]],
                    rows_v.at[pl.ds(j * LANES, LANES), :],
                    sem,
                ).wait()

        def fire_store(step, rows_v, sem):
            pltpu.async_copy(
                rows_v,
                out_hbm.at[pl.ds(out_base + step * CG, CG), pl.ds(0, D_MODEL)],
                sem,
            )

        def wait_store(step, rows_v, sem):
            pltpu.make_async_copy(
                rows_v,
                out_hbm.at[pl.ds(out_base + step * CG, CG), pl.ds(0, D_MODEL)],
                sem,
            ).wait()

        fire_gathers(0, rows0, sem_g0)

        def body(t, carry):
            i0 = 2 * t
            wait_gathers(rows0, sem_g0)
            fire_store(i0, rows0, sem_s0)

            @pl.when(t > 0)
            def _():
                wait_store(i0 - 1, rows1, sem_s1)

            fire_gathers(i0 + 1, rows1, sem_g1)

            wait_gathers(rows1, sem_g1)
            fire_store(i0 + 1, rows1, sem_s1)
            wait_store(i0, rows0, sem_s0)

            @pl.when(t < T - 1)
            def _():
                fire_gathers(i0 + 2, rows0, sem_g0)

            return carry

        lax.fori_loop(0, T, body, 0)
        wait_store(n_steps - 1, rows1, sem_s1)

    return gather_kernel


def kernel(x, table):
    B0, B1 = x.shape
    B = B0 * B1
    V = table.shape[0]
    idx2 = (x.astype(jnp.int32) * 2).reshape(B // LANES, LANES)
    staged = _build_transpose(V)(table.T)
    table_lin = staged.reshape(2 * V, D_MODEL)
    out = _build_gather(B, 2 * V)(idx2, table_lin)
    return out[:, :D_MODEL].reshape(B0, B1, D_MODEL)


# speedup vs baseline: 1.4975x; 1.0014x over previous
import functools

import jax
import jax.numpy as jnp
from jax import lax
from jax.experimental import pallas as pl
from jax.experimental.pallas import tpu as pltpu
from jax.experimental.pallas import tpu_sc as plsc

D_MODEL = 64
WIDE = 128
VOCAB_CHUNK = 32768
LANES = 128
K = 5
NC = 2
NS = 16
NW = NC * NS


def _transpose_block(in_ref, out_ref):
    out_ref[:, 0:D_MODEL] = in_ref[...].T


@functools.lru_cache(maxsize=None)
def _build_transpose(V):
    grid = (V + VOCAB_CHUNK - 1) // VOCAB_CHUNK
    return pl.pallas_call(
        _transpose_block,
        grid=(grid,),
        in_specs=[pl.BlockSpec((D_MODEL, VOCAB_CHUNK), lambda i: (0, i))],
        out_specs=pl.BlockSpec((VOCAB_CHUNK, WIDE), lambda i: (i, 0)),
        out_shape=jax.ShapeDtypeStruct((V, WIDE), jnp.float32),
    )


@functools.lru_cache(maxsize=None)
def _build_gather(B, V2):
    rows_total = B // LANES
    rows_per_w = rows_total // NW
    n_steps = rows_per_w // K
    T = n_steps // 2
    CG = K * LANES

    mesh = plsc.VectorSubcoreMesh(core_axis_name="c", subcore_axis_name="s")

    @functools.partial(
        pl.kernel,
        mesh=mesh,
        out_type=jax.ShapeDtypeStruct((B, WIDE), jnp.float32),
        scratch_types=[
            pltpu.VMEM((rows_per_w, LANES), jnp.int32),
            pltpu.VMEM((CG, D_MODEL), jnp.float32),
            pltpu.VMEM((CG, D_MODEL), jnp.float32),
            pltpu.SemaphoreType.DMA,
            pltpu.SemaphoreType.DMA,
            pltpu.SemaphoreType.DMA,
            pltpu.SemaphoreType.DMA,
        ],
        compiler_params=pltpu.CompilerParams(use_tc_tiling_on_sc=False),
    )
    def gather_kernel(idx_hbm, table_hbm, out_hbm, idx_all, rows0, rows1,
                      sem_g0, sem_g1, sem_s0, sem_s1):
        wid = lax.axis_index("s") * NC + lax.axis_index("c")
        row_base = wid * rows_per_w
        out_base = row_base * LANES

        pltpu.sync_copy(idx_hbm.at[pl.ds(row_base, rows_per_w), :], idx_all)

        def fire_gathers(step, rows_v, sem):
            for j in range(K):
                pltpu.async_copy(
                    table_hbm.at[idx_all.at[step * K + j]],
                    rows_v.at[pl.ds(j * LANES, LANES), :],
                    sem,
                )

        def wait_gathers(rows_v, sem):
            for j in range(K):
                pltpu.make_async_copy(
                    table_hbm.at[idx_all.at[j]],
                    rows_v.at[pl.ds(j * LANES, LANES), :],
                    sem,
                ).wait()

        def fire_store(step, rows_v, sem):
            pltpu.async_copy(
                rows_v,
                out_hbm.at[pl.ds(out_base + step * CG, CG), pl.ds(0, D_MODEL)],
                sem,
            )

        def wait_store(step, rows_v, sem):
            pltpu.make_async_copy(
                rows_v,
                out_hbm.at[pl.ds(out_base + step * CG, CG), pl.ds(0, D_MODEL)],
                sem,
            ).wait()

        fire_gathers(0, rows0, sem_g0)

        def body(t, carry):
            i0 = 2 * t

            @pl.when(t > 0)
            def _():
                wait_store(i0 - 1, rows1, sem_s1)

            fire_gathers(i0 + 1, rows1, sem_g1)
            wait_gathers(rows0, sem_g0)
            fire_store(i0, rows0, sem_s0)

            wait_store(i0, rows0, sem_s0)

            @pl.when(t < T - 1)
            def _():
                fire_gathers(i0 + 2, rows0, sem_g0)

            wait_gathers(rows1, sem_g1)
            fire_store(i0 + 1, rows1, sem_s1)
            return carry

        lax.fori_loop(0, T, body, 0)
        wait_store(n_steps - 1, rows1, sem_s1)

    return gather_kernel


def kernel(x, table):
    B0, B1 = x.shape
    B = B0 * B1
    V = table.shape[0]
    idx2 = (x.astype(jnp.int32) * 2).reshape(B // LANES, LANES)
    staged = _build_transpose(V)(table.T)
    table_lin = staged.reshape(2 * V, D_MODEL)
    out = _build_gather(B, 2 * V)(idx2, table_lin)
    return out[:, :D_MODEL].reshape(B0, B1, D_MODEL)
